# Initial kernel scaffold; baseline (speedup 1.0000x reference)
import jax
import jax.numpy as jnp
from jax.experimental import pallas as pl


def kernel(z, codebook):
    raise NotImplementedError("write your pallas kernel here")


# speedup vs baseline: 1.3695x; 1.3695x over previous
import functools

import jax
import jax.numpy as jnp
from jax import lax
from jax.experimental import pallas as pl
from jax.experimental.pallas import tpu as pltpu
from jax.experimental.pallas import tpu_sc as plsc

_NUM_CODES = 8192
_DIM = 32
_BR = 256


def _argmin_body(nr, n_total, z_ref, cbt_ref, codes_ref, loss_ref, acc_ref):
    i = pl.program_id(0)
    z = z_ref[...]
    cbt = cbt_ref[...]
    dot = lax.dot_general(
        z, cbt, (((1,), (0,)), ((), ())),
        preferred_element_type=jnp.float32,
    )
    cn = jnp.sum(cbt * cbt, axis=0, keepdims=True)
    zn = jnp.sum(z * z, axis=1, keepdims=True)
    d = (zn - 2.0 * dot) + cn
    m = jnp.min(d, axis=1, keepdims=True)
    cols = lax.broadcasted_iota(jnp.int32, d.shape, 1)
    idx = jnp.min(jnp.where(d == m, cols, _NUM_CODES), axis=1, keepdims=True)
    codes_ref[0] = idx

    @pl.when(i == 0)
    def _init():
        acc_ref[...] = jnp.zeros((1, 1), jnp.float32)

    acc_ref[...] += jnp.sum(m).reshape(1, 1)

    @pl.when(i == nr - 1)
    def _fin():
        loss_ref[...] = acc_ref[...] * (0.5 / (n_total * _DIM))


def _codes_and_loss(z_flat, codebook):
    n = z_flat.shape[0]
    nr = n // _BR
    cb_t = codebook.T
    codes3, loss = pl.pallas_call(
        functools.partial(_argmin_body, nr, n),
        grid=(nr,),
        in_specs=[
            pl.BlockSpec((_BR, _DIM), lambda i: (i, 0)),
            pl.BlockSpec((_DIM, _NUM_CODES), lambda i: (0, 0)),
        ],
        out_specs=[
            pl.BlockSpec((1, _BR, 1), lambda i: (i, 0, 0)),
            pl.BlockSpec((1, 1), lambda i: (0, 0)),
        ],
        out_shape=[
            jax.ShapeDtypeStruct((nr, _BR, 1), jnp.int32),
            jax.ShapeDtypeStruct((1, 1), jnp.float32),
        ],
        scratch_shapes=[pltpu.VMEM((1, 1), jnp.float32)],
    )(z_flat, cb_t)
    return codes3.reshape(-1), loss.reshape(())


_NC = 2
_NS = 16
_NW = _NC * _NS
_GCHUNK = 128


def _make_gather(n):
    b_per_w = n // _NW
    nchunk = b_per_w // _GCHUNK
    mesh = plsc.VectorSubcoreMesh(core_axis_name="c", subcore_axis_name="s")

    @functools.partial(
        pl.kernel,
        mesh=mesh,
        out_type=jax.ShapeDtypeStruct((n, _DIM), jnp.float32),
        scratch_types=[
            pltpu.VMEM((b_per_w,), jnp.int32),
            pltpu.VMEM((b_per_w, _DIM), jnp.float32),
            pltpu.SemaphoreType.DMA,
        ],
        compiler_params=pltpu.CompilerParams(use_tc_tiling_on_sc=False),
    )
    def gather(table_hbm, idx_hbm, out_hbm, idx_v, rows_v, sem):
        wid = lax.axis_index("s") * _NC + lax.axis_index("c")
        base = wid * b_per_w
        pltpu.sync_copy(idx_hbm.at[pl.ds(base, b_per_w)], idx_v)
        copies = []
        for t in range(nchunk):
            copies.append(pltpu.async_copy(
                table_hbm.at[idx_v.at[pl.ds(t * _GCHUNK, _GCHUNK)]],
                rows_v.at[pl.ds(t * _GCHUNK, _GCHUNK)],
                sem,
            ))
        for c in copies:
            c.wait()
        pltpu.sync_copy(rows_v, out_hbm.at[pl.ds(base, b_per_w)])

    return gather


def kernel(z, codebook):
    b, c, p, d = z.shape
    z_flat = z.reshape(-1, d)
    codes_flat, total_loss = _codes_and_loss(z_flat, codebook)
    z_q = _make_gather(z_flat.shape[0])(codebook, codes_flat)
    z_q_st = z_q.reshape(z.shape)
    codes = codes_flat.reshape(b, c, p)
    return (z_q_st, total_loss, codes)
